# Initial kernel scaffold; baseline (speedup 1.0000x reference)
import jax
import jax.numpy as jnp
from jax.experimental import pallas as pl


def kernel(x, edge_index, W_l, b_l, W_r, W_head, b_head):
    raise NotImplementedError("write your pallas kernel here")


# speedup vs baseline: 9.4358x; 9.4358x over previous
import functools

import jax
import jax.numpy as jnp
from jax import lax
from jax.experimental import pallas as pl
from jax.experimental.pallas import tpu as pltpu
from jax.experimental.pallas import tpu_sc as plsc

N, E, D, H, O = 10000, 320000, 128, 64, 2
NC, NS = 2, 16
NW = NC * NS
EPW = E // NW
CHUNK = 80
NCHUNK = EPW // CHUNK
DEG_T = 10
DEG_R = N // DEG_T


def _project_body(x_ref, wl_ref, wr_ref, bl_ref, y_ref, r_ref):
    xb = x_ref[...]
    y_ref[...] = jnp.dot(xb, wl_ref[...], preferred_element_type=jnp.float32)
    r_ref[...] = (
        jnp.dot(xb, wr_ref[...], preferred_element_type=jnp.float32)
        + bl_ref[...][None, :]
    )


def _project(x, W_l, W_r, b_l):
    BN = 2000
    return pl.pallas_call(
        _project_body,
        grid=(N // BN,),
        in_specs=[
            pl.BlockSpec((BN, D), lambda i: (i, 0)),
            pl.BlockSpec((D, H), lambda i: (0, 0)),
            pl.BlockSpec((D, H), lambda i: (0, 0)),
            pl.BlockSpec((H,), lambda i: (0,)),
        ],
        out_specs=[
            pl.BlockSpec((BN, H), lambda i: (i, 0)),
            pl.BlockSpec((BN, H), lambda i: (i, 0)),
        ],
        out_shape=[
            jax.ShapeDtypeStruct((N, H), jnp.float32),
            jax.ShapeDtypeStruct((N, H), jnp.float32),
        ],
    )(x, W_l, W_r, b_l)


def _sc_body(y_hbm, src_hbm, dst_hbm, z2_hbm, z1_hbm,
             acc_out, deg_out,
             src_v, dst_v, rows_v, ones_v, dbuf_v, acc_sh, deg_sh, sem):
    cid = lax.axis_index("c")
    sid = lax.axis_index("s")
    wid = sid * NC + cid

    @pl.when(sid < DEG_T)
    def _():
        pltpu.sync_copy(z2_hbm.at[pl.ds(sid * DEG_R, DEG_R)],
                        acc_sh.at[pl.ds(sid * DEG_R, DEG_R)])
        pltpu.sync_copy(z1_hbm.at[pl.ds(sid * DEG_R, DEG_R)], dbuf_v)
        pltpu.sync_copy(dbuf_v, deg_sh.at[pl.ds(sid * DEG_R, DEG_R)])

    pltpu.sync_copy(src_hbm.at[wid], src_v)
    pltpu.sync_copy(dst_hbm.at[wid], dst_v)
    for j in range(CHUNK // 16):
        ones_v[pl.ds(16 * j, 16)] = jnp.full((16,), 1.0, dtype=jnp.float32)

    plsc.subcore_barrier()

    def step(c, carry):
        pltpu.async_copy(y_hbm.at[src_v.at[c]], rows_v, sem).wait()
        pltpu.sync_copy(rows_v, acc_sh.at[dst_v.at[c]], add=True)
        pltpu.sync_copy(ones_v, deg_sh.at[dst_v.at[c]], add=True)
        return carry

    lax.fori_loop(0, NCHUNK, step, 0)

    plsc.subcore_barrier()

    @pl.when(sid < DEG_T)
    def _():
        pltpu.sync_copy(acc_sh.at[pl.ds(sid * DEG_R, DEG_R)],
                        acc_out.at[cid, pl.ds(sid * DEG_R, DEG_R)])
        pltpu.sync_copy(deg_sh.at[pl.ds(sid * DEG_R, DEG_R)], dbuf_v)
        pltpu.sync_copy(dbuf_v, deg_out.at[pl.ds(cid * N + sid * DEG_R, DEG_R)])


def _sc_aggregate(y, src, dst, z2, z1):
    mesh = plsc.VectorSubcoreMesh(core_axis_name="c", subcore_axis_name="s")
    f = pl.kernel(
        _sc_body,
        out_type=(
            jax.ShapeDtypeStruct((NC, N, H), jnp.float32),
            jax.ShapeDtypeStruct((NC * N,), jnp.float32),
        ),
        mesh=mesh,
        compiler_params=pltpu.CompilerParams(use_tc_tiling_on_sc=False),
        scratch_types=[
            pltpu.VMEM((NCHUNK, CHUNK), jnp.int32),
            pltpu.VMEM((NCHUNK, CHUNK), jnp.int32),
            pltpu.VMEM((CHUNK, H), jnp.float32),
            pltpu.VMEM((CHUNK,), jnp.float32),
            pltpu.VMEM((DEG_R,), jnp.float32),
            pltpu.VMEM_SHARED((N, H), jnp.float32),
            pltpu.VMEM_SHARED((N,), jnp.float32),
            pltpu.SemaphoreType.DMA,
        ],
    )
    return f(y, src, dst, z2, z1)


def _head_body(accp_ref, degp_ref, r_ref, wh_ref, bh_ref, out_ref):
    a = accp_ref[0] + accp_ref[1]
    dsum = degp_ref[0] + degp_ref[1]
    scale = 1.0 / jnp.maximum(dsum, 1.0)
    z = jnp.maximum(a * scale + r_ref[...], 0.0)
    out_ref[...] = (
        jnp.dot(z, wh_ref[...], preferred_element_type=jnp.float32)
        + bh_ref[...][None, :]
    )


def _head(accp, degp, r, W_head, b_head):
    BN = 2000
    return pl.pallas_call(
        _head_body,
        grid=(N // BN,),
        in_specs=[
            pl.BlockSpec((NC, BN, H), lambda i: (0, i, 0)),
            pl.BlockSpec((NC, BN, 1), lambda i: (0, i, 0)),
            pl.BlockSpec((BN, H), lambda i: (i, 0)),
            pl.BlockSpec((H, O), lambda i: (0, 0)),
            pl.BlockSpec((O,), lambda i: (0,)),
        ],
        out_specs=pl.BlockSpec((BN, O), lambda i: (i, 0)),
        out_shape=jax.ShapeDtypeStruct((N, O), jnp.float32),
    )(accp, degp, r, W_head, b_head)


def kernel(x, edge_index, W_l, b_l, W_r, W_head, b_head):
    src = edge_index[0].reshape(NW, NCHUNK, CHUNK)
    dst = edge_index[1].reshape(NW, NCHUNK, CHUNK)
    y, r = _project(x, W_l, W_r, b_l)
    z2 = jnp.zeros((N, H), jnp.float32)
    z1 = jnp.zeros((N,), jnp.float32)
    accp, degp = _sc_aggregate(y, src, dst, z2, z1)
    out = _head(accp, degp.reshape(NC, N, 1), r, W_head, b_head)
    return out


# speedup vs baseline: 12.5985x; 1.3352x over previous
import functools

import jax
import jax.numpy as jnp
from jax import lax
from jax.experimental import pallas as pl
from jax.experimental.pallas import tpu as pltpu
from jax.experimental.pallas import tpu_sc as plsc

N, E, D, H, O = 10000, 320000, 128, 64, 2
NC, NS = 2, 16
NW = NC * NS
EPW = E // NW
CHUNK = 125
NCHUNK = EPW // CHUNK
DEG_T = 10
DEG_R = N // DEG_T


def _project_body(x_ref, wl_ref, wr_ref, bl_ref, y_ref, r_ref):
    xb = x_ref[...]
    y_ref[...] = jnp.dot(xb, wl_ref[...], preferred_element_type=jnp.float32)
    r_ref[...] = (
        jnp.dot(xb, wr_ref[...], preferred_element_type=jnp.float32)
        + bl_ref[...][None, :]
    )


def _project(x, W_l, W_r, b_l):
    BN = 2000
    return pl.pallas_call(
        _project_body,
        grid=(N // BN,),
        in_specs=[
            pl.BlockSpec((BN, D), lambda i: (i, 0)),
            pl.BlockSpec((D, H), lambda i: (0, 0)),
            pl.BlockSpec((D, H), lambda i: (0, 0)),
            pl.BlockSpec((H,), lambda i: (0,)),
        ],
        out_specs=[
            pl.BlockSpec((BN, H), lambda i: (i, 0)),
            pl.BlockSpec((BN, H), lambda i: (i, 0)),
        ],
        out_shape=[
            jax.ShapeDtypeStruct((N, H), jnp.float32),
            jax.ShapeDtypeStruct((N, H), jnp.float32),
        ],
    )(x, W_l, W_r, b_l)


def _sc_body(y_hbm, src_hbm, dst_hbm, z2_hbm, z1_hbm,
             acc_out, deg_out,
             src_v, dst_v, rows0_v, rows1_v, ones_v, dbuf_v, acc_sh, deg_sh,
             gsem0, gsem1, ssem):
    cid = lax.axis_index("c")
    sid = lax.axis_index("s")
    wid = sid * NC + cid

    @pl.when(sid < DEG_T)
    def _():
        pltpu.sync_copy(z2_hbm.at[pl.ds(sid * DEG_R, DEG_R)],
                        acc_sh.at[pl.ds(sid * DEG_R, DEG_R)])
        pltpu.sync_copy(z1_hbm.at[pl.ds(sid * DEG_R, DEG_R)], dbuf_v)
        pltpu.sync_copy(dbuf_v, deg_sh.at[pl.ds(sid * DEG_R, DEG_R)])

    pltpu.sync_copy(src_hbm.at[wid], src_v)
    pltpu.sync_copy(dst_hbm.at[wid], dst_v)
    for j in range(8):
        ones_v[pl.ds(16 * j, 16)] = jnp.full((16,), 1.0, dtype=jnp.float32)

    plsc.subcore_barrier()

    def step(j, carry):
        c0 = 2 * j
        c1 = c0 + 1
        g0 = pltpu.async_copy(y_hbm.at[src_v.at[c0]], rows0_v, gsem0)
        g1 = pltpu.async_copy(y_hbm.at[src_v.at[c1]], rows1_v, gsem1)
        g0.wait()
        s0 = pltpu.async_copy(rows0_v, acc_sh.at[dst_v.at[c0]], ssem,
                              add=True)
        d0 = pltpu.async_copy(ones_v.at[pl.ds(0, CHUNK)],
                              deg_sh.at[dst_v.at[c0]], ssem, add=True)
        g1.wait()
        s1 = pltpu.async_copy(rows1_v, acc_sh.at[dst_v.at[c1]], ssem,
                              add=True)
        d1 = pltpu.async_copy(ones_v.at[pl.ds(0, CHUNK)],
                              deg_sh.at[dst_v.at[c1]], ssem, add=True)
        s0.wait()
        d0.wait()
        s1.wait()
        d1.wait()
        return carry

    lax.fori_loop(0, NCHUNK // 2, step, 0)

    plsc.subcore_barrier()

    @pl.when(sid < DEG_T)
    def _():
        pltpu.sync_copy(acc_sh.at[pl.ds(sid * DEG_R, DEG_R)],
                        acc_out.at[cid, pl.ds(sid * DEG_R, DEG_R)])
        pltpu.sync_copy(deg_sh.at[pl.ds(sid * DEG_R, DEG_R)], dbuf_v)
        pltpu.sync_copy(dbuf_v, deg_out.at[pl.ds(cid * N + sid * DEG_R, DEG_R)])


def _sc_aggregate(y, src, dst, z2, z1):
    mesh = plsc.VectorSubcoreMesh(core_axis_name="c", subcore_axis_name="s")
    f = pl.kernel(
        _sc_body,
        out_type=(
            jax.ShapeDtypeStruct((NC, N, H), jnp.float32),
            jax.ShapeDtypeStruct((NC * N,), jnp.float32),
        ),
        mesh=mesh,
        compiler_params=pltpu.CompilerParams(use_tc_tiling_on_sc=False),
        scratch_types=[
            pltpu.VMEM((NCHUNK, CHUNK), jnp.int32),
            pltpu.VMEM((NCHUNK, CHUNK), jnp.int32),
            pltpu.VMEM((CHUNK, H), jnp.float32),
            pltpu.VMEM((CHUNK, H), jnp.float32),
            pltpu.VMEM((128,), jnp.float32),
            pltpu.VMEM((DEG_R,), jnp.float32),
            pltpu.VMEM_SHARED((N, H), jnp.float32),
            pltpu.VMEM_SHARED((N,), jnp.float32),
            pltpu.SemaphoreType.DMA,
            pltpu.SemaphoreType.DMA,
            pltpu.SemaphoreType.DMA,
        ],
    )
    return f(y, src, dst, z2, z1)


def _head_body(accp_ref, degp_ref, r_ref, wh_ref, bh_ref, out_ref):
    a = accp_ref[0] + accp_ref[1]
    dsum = degp_ref[0] + degp_ref[1]
    scale = 1.0 / jnp.maximum(dsum, 1.0)
    z = jnp.maximum(a * scale + r_ref[...], 0.0)
    out_ref[...] = (
        jnp.dot(z, wh_ref[...], preferred_element_type=jnp.float32)
        + bh_ref[...][None, :]
    )


def _head(accp, degp, r, W_head, b_head):
    BN = 2000
    return pl.pallas_call(
        _head_body,
        grid=(N // BN,),
        in_specs=[
            pl.BlockSpec((NC, BN, H), lambda i: (0, i, 0)),
            pl.BlockSpec((NC, BN, 1), lambda i: (0, i, 0)),
            pl.BlockSpec((BN, H), lambda i: (i, 0)),
            pl.BlockSpec((H, O), lambda i: (0, 0)),
            pl.BlockSpec((O,), lambda i: (0,)),
        ],
        out_specs=pl.BlockSpec((BN, O), lambda i: (i, 0)),
        out_shape=jax.ShapeDtypeStruct((N, O), jnp.float32),
    )(accp, degp, r, W_head, b_head)


def kernel(x, edge_index, W_l, b_l, W_r, W_head, b_head):
    src = edge_index[0].reshape(NW, NCHUNK, CHUNK)
    dst = edge_index[1].reshape(NW, NCHUNK, CHUNK)
    y, r = _project(x, W_l, W_r, b_l)
    z2 = jnp.zeros((N, H), jnp.float32)
    z1 = jnp.zeros((N,), jnp.float32)
    accp, degp = _sc_aggregate(y, src, dst, z2, z1)
    out = _head(accp, degp.reshape(NC, N, 1), r, W_head, b_head)
    return out


# speedup vs baseline: 15.5295x; 1.2326x over previous
import functools

import jax
import jax.numpy as jnp
from jax import lax
from jax.experimental import pallas as pl
from jax.experimental.pallas import tpu as pltpu
from jax.experimental.pallas import tpu_sc as plsc

N, E, D, H, O = 10000, 320000, 128, 64, 2
NC, NS = 2, 16
NW = NC * NS
EPW = E // NW
CHUNK = 125
NCHUNK = EPW // CHUNK
DEG_T = 10
DEG_R = N // DEG_T


def _project_body(x_ref, wl_ref, wr_ref, bl_ref, y_ref, r_ref):
    xb = x_ref[...]
    y_ref[...] = jnp.dot(xb, wl_ref[...], preferred_element_type=jnp.float32)
    r_ref[...] = (
        jnp.dot(xb, wr_ref[...], preferred_element_type=jnp.float32)
        + bl_ref[...][None, :]
    )


def _project(x, W_l, W_r, b_l):
    BN = 2000
    return pl.pallas_call(
        _project_body,
        grid=(N // BN,),
        in_specs=[
            pl.BlockSpec((BN, D), lambda i: (i, 0)),
            pl.BlockSpec((D, H), lambda i: (0, 0)),
            pl.BlockSpec((D, H), lambda i: (0, 0)),
            pl.BlockSpec((H,), lambda i: (0,)),
        ],
        out_specs=[
            pl.BlockSpec((BN, H), lambda i: (i, 0)),
            pl.BlockSpec((BN, H), lambda i: (i, 0)),
        ],
        out_shape=[
            jax.ShapeDtypeStruct((N, H), jnp.float32),
            jax.ShapeDtypeStruct((N, H), jnp.float32),
        ],
    )(x, W_l, W_r, b_l)


NBUF = 4


def _sc_body(y_hbm, edges_hbm, z2_hbm, z1_hbm,
             acc_out, deg_out,
             src_v, dst_v, rows_v, ones_v, dbuf_v, acc_sh, deg_sh,
             gsem0, gsem1, gsem2, gsem3,
             ssem0, ssem1, ssem2, ssem3,
             dsem0, dsem1, dsem2, dsem3):
    gsems = (gsem0, gsem1, gsem2, gsem3)
    ssems = (ssem0, ssem1, ssem2, ssem3)
    dsems = (dsem0, dsem1, dsem2, dsem3)
    cid = lax.axis_index("c")
    sid = lax.axis_index("s")
    wid = sid * NC + cid

    @pl.when(sid < DEG_T)
    def _():
        pltpu.sync_copy(z2_hbm.at[pl.ds(sid * DEG_R, DEG_R)],
                        acc_sh.at[pl.ds(sid * DEG_R, DEG_R)])
        pltpu.sync_copy(z1_hbm.at[pl.ds(sid * DEG_R, DEG_R)], dbuf_v)
        pltpu.sync_copy(dbuf_v, deg_sh.at[pl.ds(sid * DEG_R, DEG_R)])

    pltpu.sync_copy(edges_hbm.at[0, wid], src_v)
    pltpu.sync_copy(edges_hbm.at[1, wid], dst_v)
    for j in range(8):
        ones_v[pl.ds(16 * j, 16)] = jnp.full((16,), 1.0, dtype=jnp.float32)

    plsc.subcore_barrier()

    pltpu.async_copy(y_hbm.at[src_v.at[0]], rows_v.at[0], gsems[0])
    pltpu.async_copy(y_hbm.at[src_v.at[1]], rows_v.at[1], gsems[1])

    def group(g, carry):
        for b in range(NBUF):
            c = NBUF * g + b
            b2 = (b + 2) % NBUF

            pltpu.make_async_copy(y_hbm.at[src_v.at[c]], rows_v.at[b],
                                  gsems[b]).wait()
            pltpu.async_copy(rows_v.at[b], acc_sh.at[dst_v.at[c]],
                             ssems[b], add=True)
            pltpu.async_copy(ones_v.at[pl.ds(0, CHUNK)],
                             deg_sh.at[dst_v.at[c]], dsems[b], add=True)

            @pl.when(c >= 2)
            def _():
                pltpu.make_async_copy(rows_v.at[b2],
                                      acc_sh.at[dst_v.at[c - 2]],
                                      ssems[b2]).wait()
                pltpu.make_async_copy(ones_v.at[pl.ds(0, CHUNK)],
                                      deg_sh.at[dst_v.at[c - 2]],
                                      dsems[b2]).wait()

            @pl.when(c < NCHUNK - 2)
            def _():
                pltpu.async_copy(y_hbm.at[src_v.at[c + 2]], rows_v.at[b2],
                                 gsems[b2])
        return carry

    lax.fori_loop(0, NCHUNK // NBUF, group, 0)

    pltpu.make_async_copy(rows_v.at[2], acc_sh.at[dst_v.at[NCHUNK - 2]],
                          ssems[2]).wait()
    pltpu.make_async_copy(ones_v.at[pl.ds(0, CHUNK)],
                          deg_sh.at[dst_v.at[NCHUNK - 2]], dsems[2]).wait()
    pltpu.make_async_copy(rows_v.at[3], acc_sh.at[dst_v.at[NCHUNK - 1]],
                          ssems[3]).wait()
    pltpu.make_async_copy(ones_v.at[pl.ds(0, CHUNK)],
                          deg_sh.at[dst_v.at[NCHUNK - 1]], dsems[3]).wait()

    plsc.subcore_barrier()

    @pl.when(sid < DEG_T)
    def _():
        pltpu.sync_copy(acc_sh.at[pl.ds(sid * DEG_R, DEG_R)],
                        acc_out.at[cid, pl.ds(sid * DEG_R, DEG_R)])
        pltpu.sync_copy(deg_sh.at[pl.ds(sid * DEG_R, DEG_R)], dbuf_v)
        pltpu.sync_copy(dbuf_v, deg_out.at[pl.ds(cid * N + sid * DEG_R, DEG_R)])


def _sc_aggregate(y, edges, z2, z1):
    mesh = plsc.VectorSubcoreMesh(core_axis_name="c", subcore_axis_name="s")
    f = pl.kernel(
        _sc_body,
        out_type=(
            jax.ShapeDtypeStruct((NC, N, H), jnp.float32),
            jax.ShapeDtypeStruct((NC * N,), jnp.float32),
        ),
        mesh=mesh,
        compiler_params=pltpu.CompilerParams(use_tc_tiling_on_sc=False),
        scratch_types=[
            pltpu.VMEM((NCHUNK, CHUNK), jnp.int32),
            pltpu.VMEM((NCHUNK, CHUNK), jnp.int32),
            pltpu.VMEM((NBUF, CHUNK, H), jnp.float32),
            pltpu.VMEM((128,), jnp.float32),
            pltpu.VMEM((DEG_R,), jnp.float32),
            pltpu.VMEM_SHARED((N, H), jnp.float32),
            pltpu.VMEM_SHARED((N,), jnp.float32),
        ] + [pltpu.SemaphoreType.DMA] * 12,
    )
    return f(y, edges, z2, z1)


def _head_body(accp_ref, degp_ref, r_ref, wh_ref, bh_ref, out_ref):
    a = accp_ref[0] + accp_ref[1]
    dsum = degp_ref[0] + degp_ref[1]
    scale = 1.0 / jnp.maximum(dsum, 1.0)
    z = jnp.maximum(a * scale + r_ref[...], 0.0)
    out_ref[...] = (
        jnp.dot(z, wh_ref[...], preferred_element_type=jnp.float32)
        + bh_ref[...][None, :]
    )


def _head(accp, degp, r, W_head, b_head):
    BN = 2000
    return pl.pallas_call(
        _head_body,
        grid=(N // BN,),
        in_specs=[
            pl.BlockSpec((NC, BN, H), lambda i: (0, i, 0)),
            pl.BlockSpec((NC, BN, 1), lambda i: (0, i, 0)),
            pl.BlockSpec((BN, H), lambda i: (i, 0)),
            pl.BlockSpec((H, O), lambda i: (0, 0)),
            pl.BlockSpec((O,), lambda i: (0,)),
        ],
        out_specs=pl.BlockSpec((BN, O), lambda i: (i, 0)),
        out_shape=jax.ShapeDtypeStruct((N, O), jnp.float32),
    )(accp, degp, r, W_head, b_head)


def kernel(x, edge_index, W_l, b_l, W_r, W_head, b_head):
    edges = edge_index.reshape(2, NW, NCHUNK, CHUNK)
    y, r = _project(x, W_l, W_r, b_l)
    z2 = jnp.zeros((N, H), jnp.float32)
    z1 = jnp.zeros((N,), jnp.float32)
    accp, degp = _sc_aggregate(y, edges, z2, z1)
    out = _head(accp, degp.reshape(NC, N, 1), r, W_head, b_head)
    return out
